# Initial kernel scaffold; baseline (speedup 1.0000x reference)
import jax
import jax.numpy as jnp
from jax.experimental import pallas as pl


def kernel(h, idx0_onefour, idx1_onefour, idx0_nonbonded, idx1_nonbonded, W1_of, b1_of, Wsig_of, bsig_of, Weps_of, beps_of, W1_nb, b1_nb, Wsig_nb, bsig_nb, Weps_nb, beps_nb):
    raise NotImplementedError("write your pallas kernel here")


# speedup vs baseline: 1.8949x; 1.8949x over previous
import functools

import jax
import jax.numpy as jnp
from jax import lax
from jax.experimental import pallas as pl
from jax.experimental.pallas import tpu as pltpu
from jax.experimental.pallas import tpu_sc as plsc

N = 10000
D = 128
H = 16
CHUNK = 128


def _tables_body(h_ref, wof_ref, bof_ref, wnb_ref, bnb_ref, tof_ref, tnb_ref):
    h = h_ref[...]
    tof_ref[...] = jnp.dot(h, wof_ref[...], preferred_element_type=jnp.float32) + bof_ref[...]
    tnb_ref[...] = jnp.dot(h, wnb_ref[...], preferred_element_type=jnp.float32) + bnb_ref[...]


def _make_tables(h, wof, bof, wnb, bnb):
    return pl.pallas_call(
        _tables_body,
        out_shape=[
            jax.ShapeDtypeStruct((N, 2 * H), jnp.float32),
            jax.ShapeDtypeStruct((N, 2 * H), jnp.float32),
        ],
    )(h, wof, bof, wnb, bnb)


def _make_gather(ppad):
    info = plsc.get_sparse_core_info()
    nc, ns = info.num_cores, info.num_subcores
    nw = nc * ns
    per_tile = ppad // nw
    assert per_tile % CHUNK == 0
    chunks = per_tile // CHUNK
    mesh = plsc.VectorSubcoreMesh(core_axis_name="c", subcore_axis_name="s")

    @functools.partial(
        pl.kernel,
        mesh=mesh,
        out_type=[
            jax.ShapeDtypeStruct((ppad, 2 * H), jnp.float32),
            jax.ShapeDtypeStruct((ppad, 2 * H), jnp.float32),
        ],
        scratch_types=[
            pltpu.VMEM((CHUNK,), jnp.int32),
            pltpu.VMEM((CHUNK,), jnp.int32),
            pltpu.VMEM((CHUNK, 2 * H), jnp.float32),
            pltpu.VMEM((CHUNK, 2 * H), jnp.float32),
            pltpu.SemaphoreType.DMA,
            pltpu.SemaphoreType.DMA,
        ],
        compiler_params=pltpu.CompilerParams(use_tc_tiling_on_sc=False),
    )
    def k(t_hbm, idx0_hbm, idx1_hbm, g0_hbm, g1_hbm,
          idx0_v, idx1_v, rows0_v, rows1_v, sem0, sem1):
        wid = lax.axis_index("s") * nc + lax.axis_index("c")

        def chunk(c, carry):
            base = wid * per_tile + c * CHUNK
            pltpu.sync_copy(idx0_hbm.at[pl.ds(base, CHUNK)], idx0_v)
            pltpu.sync_copy(idx1_hbm.at[pl.ds(base, CHUNK)], idx1_v)
            cp0 = pltpu.async_copy(t_hbm.at[idx0_v], rows0_v, sem0)
            cp1 = pltpu.async_copy(t_hbm.at[idx1_v], rows1_v, sem1)
            cp0.wait()
            cp1.wait()
            pltpu.sync_copy(rows0_v, g0_hbm.at[pl.ds(base, CHUNK)])
            pltpu.sync_copy(rows1_v, g1_hbm.at[pl.ds(base, CHUNK)])
            return carry

        lax.fori_loop(0, chunks, chunk, 0)

    return k


def _finish_body(g0_ref, g1_ref, wh_ref, bh_ref, out_ref):
    g0 = g0_ref[...]
    g1 = g1_ref[...]
    s = (jnp.maximum(g0[:, :H] + g1[:, H:], 0.0)
         + jnp.maximum(g1[:, :H] + g0[:, H:], 0.0))
    out_ref[...] = jnp.dot(s, wh_ref[...], preferred_element_type=jnp.float32) + bh_ref[...]


def _finish(g0, g1, wh, bh, ppad, blk):
    return pl.pallas_call(
        _finish_body,
        grid=(ppad // blk,),
        in_specs=[
            pl.BlockSpec((blk, 2 * H), lambda i: (i, 0)),
            pl.BlockSpec((blk, 2 * H), lambda i: (i, 0)),
            pl.BlockSpec((H, 2), lambda i: (0, 0)),
            pl.BlockSpec((1, 2), lambda i: (0, 0)),
        ],
        out_specs=pl.BlockSpec((blk, 2), lambda i: (i, 0)),
        out_shape=jax.ShapeDtypeStruct((ppad, 2), jnp.float32),
    )(g0, g1, wh, bh)


def _pad_idx(idx, ppad):
    p = idx.shape[0]
    if p == ppad:
        return idx
    return jnp.concatenate([idx, jnp.zeros((ppad - p,), jnp.int32)])


def kernel(h, idx0_onefour, idx1_onefour, idx0_nonbonded, idx1_nonbonded,
           W1_of, b1_of, Wsig_of, bsig_of, Weps_of, beps_of,
           W1_nb, b1_nb, Wsig_nb, bsig_nb, Weps_nb, beps_nb):
    wof = jnp.concatenate([W1_of[:D], W1_of[D:]], axis=1)
    wnb = jnp.concatenate([W1_nb[:D], W1_nb[D:]], axis=1)
    bof = jnp.concatenate([jnp.zeros((H,), jnp.float32), b1_of]).reshape(1, 2 * H)
    bnb = jnp.concatenate([jnp.zeros((H,), jnp.float32), b1_nb]).reshape(1, 2 * H)
    wh_of = jnp.concatenate([Wsig_of, Weps_of], axis=1)
    wh_nb = jnp.concatenate([Wsig_nb, Weps_nb], axis=1)
    bh_of = jnp.concatenate([bsig_of, beps_of]).reshape(1, 2)
    bh_nb = jnp.concatenate([bsig_nb, beps_nb]).reshape(1, 2)

    t_of, t_nb = _make_tables(h, wof, bof, wnb, bnb)

    p_of = idx0_onefour.shape[0]
    p_nb = idx0_nonbonded.shape[0]
    gran = 32 * CHUNK
    ppad_of = ((p_of + gran - 1) // gran) * gran
    ppad_nb = ((p_nb + gran - 1) // gran) * gran

    g0_of, g1_of = _make_gather(ppad_of)(
        t_of, _pad_idx(idx0_onefour, ppad_of), _pad_idx(idx1_onefour, ppad_of))
    g0_nb, g1_nb = _make_gather(ppad_nb)(
        t_nb, _pad_idx(idx0_nonbonded, ppad_nb), _pad_idx(idx1_nonbonded, ppad_nb))

    out_of = _finish(g0_of, g1_of, wh_of, bh_of, ppad_of, 4096)[:p_of]
    out_nb = _finish(g0_nb, g1_nb, wh_nb, bh_nb, ppad_nb, 4096)[:p_nb]
    return (out_of, out_nb)
